# baseline (speedup 1.0000x reference)
import jax
import jax.numpy as jnp
from jax import lax
from jax.experimental import pallas as pl
from jax.experimental.pallas import tpu as pltpu
from jax.experimental.pallas import tpu_sc as plsc

_NUM_NODES = 200000
_H = 512
_W = 512
_BETA_F = 100.0

_PADH = 100352
_ROWS = _PADH // 128
_GRID = 2
_BLK = _ROWS // _GRID
_NC, _NS = 2, 16
_NW = _NC * _NS
_B = _H * _W
_BPW = _B // _NW
_RPW = _H // _NW
_LANES = 16


def _pack_table_body(t1, t2, t3, lo, hi, out):
    def nv(x):
        m = jnp.minimum(
            jnp.minimum(x[0] - t1[0, 0], x[1] - t2[0, 0]),
            x[2] - t3[0, 0],
        )
        v = jax.nn.sigmoid(_BETA_F * m) * x[3]
        bits = lax.bitcast_convert_type(v, jnp.int32)
        return (bits + 0x7FFF + ((bits >> 16) & 1)) >> 16

    out[...] = nv(lo[...]) | (nv(hi[...]) << 16)


def _gather_body(table, idx, out, table_v, idx_v, vals_v, sem):
    wid = lax.axis_index("s") * _NC + lax.axis_index("c")

    @pl.when(wid < _NW // 2)
    def _active():
        chunk = _PADH // 4
        copies = [
            pltpu.make_async_copy(table.at[pl.ds(k * chunk, chunk)],
                                  table_v.at[pl.ds(k * chunk, chunk)], sem)
            for k in range(4)
        ]
        for c in copies:
            c.start()
        for c in copies:
            c.wait()

        def do_slice(sl):
            pltpu.sync_copy(idx.at[pl.ds(sl * _BPW, _BPW)], idx_v)

            @plsc.parallel_loop(0, _BPW // _LANES, 1, unroll=16)
            def _gather_loop(i):
                off = i * _LANES
                iv = idx_v[pl.ds(off, _LANES)]
                hi = iv >= _PADH
                word_idx = iv - jnp.where(hi, _PADH, 0)
                w = plsc.load_gather(table_v, [word_idx])
                fbits = (w >> jnp.where(hi, 16, 0)) << 16
                vals_v[pl.ds(off, _LANES)] = plsc.bitcast(fbits, jnp.float32)

            for r in range(_RPW):
                pltpu.sync_copy(vals_v.at[pl.ds(r * _W, _W)],
                                out.at[sl * _RPW + r, :])

        do_slice(wid)
        do_slice(wid + _NW // 2)


def kernel(a_scaled_1, a_scaled_2, a_scaled_3, thr_1, thr_2, thr_3,
           node_levels, pixel_to_node):
    stacked = jnp.stack([a_scaled_1, a_scaled_2, a_scaled_3, node_levels])
    stacked = jnp.pad(stacked, ((0, 0), (0, 2 * _PADH - _NUM_NODES)))
    stacked = stacked.reshape(4, 2 * _ROWS, 128)
    t1 = thr_1.reshape(1, 1)
    t2 = thr_2.reshape(1, 1)
    t3 = thr_3.reshape(1, 1)

    smem = pl.BlockSpec((1, 1), lambda i: (0, 0), memory_space=pltpu.SMEM)
    lo_spec = pl.BlockSpec((4, _BLK, 128), lambda i: (0, i, 0))
    hi_spec = pl.BlockSpec((4, _BLK, 128), lambda i: (0, _GRID + i, 0))
    table = pl.pallas_call(
        _pack_table_body,
        grid=(_GRID,),
        out_shape=jax.ShapeDtypeStruct((_ROWS, 128), jnp.int32),
        in_specs=[smem, smem, smem, lo_spec, hi_spec],
        out_specs=pl.BlockSpec((_BLK, 128), lambda i: (i, 0)),
    )(t1, t2, t3, stacked, stacked).reshape(-1)

    gk = pl.kernel(
        _gather_body,
        out_type=jax.ShapeDtypeStruct((_H, _W), jnp.float32),
        mesh=plsc.VectorSubcoreMesh(core_axis_name="c", subcore_axis_name="s"),
        compiler_params=pltpu.CompilerParams(needs_layout_passes=False),
        scratch_types=[
            pltpu.VMEM((_PADH,), jnp.int32),
            pltpu.VMEM((_BPW,), jnp.int32),
            pltpu.VMEM((_BPW,), jnp.float32),
            pltpu.SemaphoreType.DMA,
        ],
    )
    return gk(table, pixel_to_node)


# speedup vs baseline: 1.5064x; 1.5064x over previous
import jax
import jax.numpy as jnp
from jax import lax
from jax.experimental import pallas as pl
from jax.experimental.pallas import tpu as pltpu
from jax.experimental.pallas import tpu_sc as plsc

_NUM_NODES = 200000
_H = 512
_W = 512
_BETA_F = 100.0

_PADH = 100352
_ROWS = _PADH // 128
_GRID = 2
_BLK = _ROWS // _GRID
_NC, _NS = 2, 16
_NW = _NC * _NS
_B = _H * _W
_BPW = _B // _NW
_RPW = _H // _NW
_LANES = 16


def _pack_table_body(t1, t2, t3, a1, a2, a3, lv, out):
    m = jnp.minimum(
        jnp.minimum(a1[...] - t1[0, 0], a2[...] - t2[0, 0]),
        a3[...] - t3[0, 0],
    )
    nv = jax.nn.sigmoid(_BETA_F * m) * lv[...]
    bits = lax.bitcast_convert_type(nv, jnp.int32)
    r = (bits + 0x7FFF + ((bits >> 16) & 1)) >> 16
    out[...] = r[:_ROWS] | (r[_ROWS:] << 16)


def _gather_body(table, idx, out, table_v, idx_v, vals_v, sem):
    wid = lax.axis_index("s") * _NC + lax.axis_index("c")

    @pl.when(wid < _NW // 2)
    def _active():
        chunk = _PADH // 4
        copies = [
            pltpu.make_async_copy(table.at[pl.ds(k * chunk, chunk)],
                                  table_v.at[pl.ds(k * chunk, chunk)], sem)
            for k in range(4)
        ]
        for c in copies:
            c.start()
        for c in copies:
            c.wait()

        def do_slice(sl):
            pltpu.sync_copy(idx.at[pl.ds(sl * _BPW, _BPW)], idx_v)

            @plsc.parallel_loop(0, _BPW // _LANES, 1, unroll=16)
            def _gather_loop(i):
                off = i * _LANES
                iv = idx_v[pl.ds(off, _LANES)]
                hi = iv >= _PADH
                word_idx = iv - jnp.where(hi, _PADH, 0)
                w = plsc.load_gather(table_v, [word_idx])
                fbits = (w >> jnp.where(hi, 16, 0)) << 16
                vals_v[pl.ds(off, _LANES)] = plsc.bitcast(fbits, jnp.float32)

            for r in range(_RPW):
                pltpu.sync_copy(vals_v.at[pl.ds(r * _W, _W)],
                                out.at[sl * _RPW + r, :])

        do_slice(wid)
        do_slice(wid + _NW // 2)


def kernel(a_scaled_1, a_scaled_2, a_scaled_3, thr_1, thr_2, thr_3,
           node_levels, pixel_to_node):
    def prep(x):
        return jnp.pad(x, (0, 2 * _PADH - _NUM_NODES)).reshape(2 * _ROWS, 128)

    a1 = prep(a_scaled_1)
    a2 = prep(a_scaled_2)
    a3 = prep(a_scaled_3)
    lv = prep(node_levels)
    t1 = thr_1.reshape(1, 1)
    t2 = thr_2.reshape(1, 1)
    t3 = thr_3.reshape(1, 1)

    smem = pl.BlockSpec(memory_space=pltpu.SMEM)
    vmem = pl.BlockSpec(memory_space=pltpu.VMEM)
    table = pl.pallas_call(
        _pack_table_body,
        out_shape=jax.ShapeDtypeStruct((_ROWS, 128), jnp.int32),
        in_specs=[smem, smem, smem, vmem, vmem, vmem, vmem],
        out_specs=vmem,
    )(t1, t2, t3, a1, a2, a3, lv).reshape(-1)

    gk = pl.kernel(
        _gather_body,
        out_type=jax.ShapeDtypeStruct((_H, _W), jnp.float32),
        mesh=plsc.VectorSubcoreMesh(core_axis_name="c", subcore_axis_name="s"),
        compiler_params=pltpu.CompilerParams(needs_layout_passes=False),
        scratch_types=[
            pltpu.VMEM((_PADH,), jnp.int32),
            pltpu.VMEM((_BPW,), jnp.int32),
            pltpu.VMEM((_BPW,), jnp.float32),
            pltpu.SemaphoreType.DMA,
        ],
    )
    return gk(table, pixel_to_node)


# speedup vs baseline: 1.5951x; 1.0589x over previous
import jax
import jax.numpy as jnp
from jax import lax
from jax.experimental import pallas as pl
from jax.experimental.pallas import tpu as pltpu
from jax.experimental.pallas import tpu_sc as plsc

_NUM_NODES = 200000
_H = 512
_W = 512
_BETA_F = 100.0

_PADH = 100352
_ROWS = _PADH // 128
_GRID = 2
_BLK = _ROWS // _GRID
_NC, _NS = 2, 16
_NW = _NC * _NS
_B = _H * _W
_BPW = _B // _NW
_RPW = _H // _NW
_LANES = 16


def _pack_table_body(t1, t2, t3, a1, a2, a3, lv, out):
    m = jnp.minimum(
        jnp.minimum(a1[...] - t1[0, 0], a2[...] - t2[0, 0]),
        a3[...] - t3[0, 0],
    )
    nv = jax.nn.sigmoid(_BETA_F * m) * lv[...]
    bits = lax.bitcast_convert_type(nv, jnp.int32)
    r = (bits + 0x7FFF + ((bits >> 16) & 1)) >> 16
    out[...] = r[:_ROWS] | (r[_ROWS:] << 16)


def _gather_body(table, idx, out, table_v, idx_v, vals_v, sem):
    wid = lax.axis_index("s") * _NC + lax.axis_index("c")

    @pl.when(wid < _NW // 2)
    def _active():
        chunk = _PADH // 4
        copies = [
            pltpu.make_async_copy(table.at[pl.ds(k * chunk, chunk)],
                                  table_v.at[pl.ds(k * chunk, chunk)], sem)
            for k in range(4)
        ]
        for c in copies:
            c.start()

        def idx_copy(sl):
            return pltpu.make_async_copy(idx.at[pl.ds(sl * _BPW, _BPW)],
                                         idx_v, sem)

        def out_copies(sl):
            return [
                pltpu.make_async_copy(vals_v.at[pl.ds(r * _W, _W)],
                                      out.at[sl * _RPW + r, :], sem)
                for r in range(_RPW)
            ]

        def gather_loop():
            @plsc.parallel_loop(0, _BPW // _LANES, 1, unroll=16)
            def _gather(i):
                off = i * _LANES
                iv = idx_v[pl.ds(off, _LANES)]
                hi = iv >= _PADH
                word_idx = iv - jnp.where(hi, _PADH, 0)
                w = plsc.load_gather(table_v, [word_idx])
                fbits = (w >> jnp.where(hi, 16, 0)) << 16
                vals_v[pl.ds(off, _LANES)] = plsc.bitcast(fbits, jnp.float32)

        sl_a = wid
        sl_b = wid + _NW // 2
        ca = idx_copy(sl_a)
        ca.start()
        for c in copies:
            c.wait()
        ca.wait()
        gather_loop()
        outs_a = out_copies(sl_a)
        for c in outs_a:
            c.start()
        cb = idx_copy(sl_b)
        cb.start()
        for c in outs_a:
            c.wait()
        cb.wait()
        gather_loop()
        outs_b = out_copies(sl_b)
        for c in outs_b:
            c.start()
        for c in outs_b:
            c.wait()


def kernel(a_scaled_1, a_scaled_2, a_scaled_3, thr_1, thr_2, thr_3,
           node_levels, pixel_to_node):
    def prep(x):
        return jnp.pad(x, (0, 2 * _PADH - _NUM_NODES)).reshape(2 * _ROWS, 128)

    a1 = prep(a_scaled_1)
    a2 = prep(a_scaled_2)
    a3 = prep(a_scaled_3)
    lv = prep(node_levels)
    t1 = thr_1.reshape(1, 1)
    t2 = thr_2.reshape(1, 1)
    t3 = thr_3.reshape(1, 1)

    smem = pl.BlockSpec(memory_space=pltpu.SMEM)
    vmem = pl.BlockSpec(memory_space=pltpu.VMEM)
    table = pl.pallas_call(
        _pack_table_body,
        out_shape=jax.ShapeDtypeStruct((_ROWS, 128), jnp.int32),
        in_specs=[smem, smem, smem, vmem, vmem, vmem, vmem],
        out_specs=vmem,
    )(t1, t2, t3, a1, a2, a3, lv).reshape(-1)

    gk = pl.kernel(
        _gather_body,
        out_type=jax.ShapeDtypeStruct((_H, _W), jnp.float32),
        mesh=plsc.VectorSubcoreMesh(core_axis_name="c", subcore_axis_name="s"),
        compiler_params=pltpu.CompilerParams(needs_layout_passes=False),
        scratch_types=[
            pltpu.VMEM((_PADH,), jnp.int32),
            pltpu.VMEM((_BPW,), jnp.int32),
            pltpu.VMEM((_BPW,), jnp.float32),
            pltpu.SemaphoreType.DMA,
        ],
    )
    return gk(table, pixel_to_node)


# speedup vs baseline: 1.6002x; 1.0032x over previous
import jax
import jax.numpy as jnp
from jax import lax
from jax.experimental import pallas as pl
from jax.experimental.pallas import tpu as pltpu
from jax.experimental.pallas import tpu_sc as plsc

_NUM_NODES = 200000
_H = 512
_W = 512
_BETA_F = 100.0

_PADH = 100352
_ROWS = _PADH // 128
_GRID = 2
_BLK = _ROWS // _GRID
_NC, _NS = 2, 16
_NW = _NC * _NS
_B = _H * _W
_BPW = _B // _NW
_RPW = _H // _NW
_LANES = 16


def _pack_table_body(t1, t2, t3, a1, a2, a3, lv, out):
    m = jnp.minimum(
        jnp.minimum(a1[...] - t1[0, 0], a2[...] - t2[0, 0]),
        a3[...] - t3[0, 0],
    )
    nv = jax.nn.sigmoid(_BETA_F * m) * lv[...]
    bits = lax.bitcast_convert_type(nv, jnp.int32)
    r = (bits + 0x7FFF + ((bits >> 16) & 1)) >> 16
    out[...] = r[:_ROWS] | (r[_ROWS:] << 16)


def _gather_body(table, idx, out, table_v, idx_v, vals_v, sem):
    wid = lax.axis_index("s") * _NC + lax.axis_index("c")

    @pl.when(wid < _NW // 2)
    def _active():
        chunk = _PADH // 8
        copies = [
            pltpu.make_async_copy(table.at[pl.ds(k * chunk, chunk)],
                                  table_v.at[pl.ds(k * chunk, chunk)], sem)
            for k in range(8)
        ]
        for c in copies:
            c.start()

        def idx_copy(sl):
            return pltpu.make_async_copy(idx.at[pl.ds(sl * _BPW, _BPW)],
                                         idx_v, sem)

        def out_copies(sl):
            return [
                pltpu.make_async_copy(vals_v.at[pl.ds(r * _W, _W)],
                                      out.at[sl * _RPW + r, :], sem)
                for r in range(_RPW)
            ]

        def gather_loop():
            @plsc.parallel_loop(0, _BPW // _LANES, 1, unroll=16)
            def _gather(i):
                off = i * _LANES
                iv = idx_v[pl.ds(off, _LANES)]
                hi = iv >= _PADH
                word_idx = iv - jnp.where(hi, _PADH, 0)
                w = plsc.load_gather(table_v, [word_idx])
                fbits = (w >> jnp.where(hi, 16, 0)) << 16
                vals_v[pl.ds(off, _LANES)] = plsc.bitcast(fbits, jnp.float32)

        sl_a = wid
        sl_b = wid + _NW // 2
        ca = idx_copy(sl_a)
        ca.start()
        for c in copies:
            c.wait()
        ca.wait()
        gather_loop()
        outs_a = out_copies(sl_a)
        for c in outs_a:
            c.start()
        cb = idx_copy(sl_b)
        cb.start()
        for c in outs_a:
            c.wait()
        cb.wait()
        gather_loop()
        outs_b = out_copies(sl_b)
        for c in outs_b:
            c.start()
        for c in outs_b:
            c.wait()


def kernel(a_scaled_1, a_scaled_2, a_scaled_3, thr_1, thr_2, thr_3,
           node_levels, pixel_to_node):
    def prep(x):
        return jnp.pad(x, (0, 2 * _PADH - _NUM_NODES)).reshape(2 * _ROWS, 128)

    a1 = prep(a_scaled_1)
    a2 = prep(a_scaled_2)
    a3 = prep(a_scaled_3)
    lv = prep(node_levels)
    t1 = thr_1.reshape(1, 1)
    t2 = thr_2.reshape(1, 1)
    t3 = thr_3.reshape(1, 1)

    smem = pl.BlockSpec(memory_space=pltpu.SMEM)
    vmem = pl.BlockSpec(memory_space=pltpu.VMEM)
    table = pl.pallas_call(
        _pack_table_body,
        out_shape=jax.ShapeDtypeStruct((_ROWS, 128), jnp.int32),
        in_specs=[smem, smem, smem, vmem, vmem, vmem, vmem],
        out_specs=vmem,
    )(t1, t2, t3, a1, a2, a3, lv).reshape(-1)

    gk = pl.kernel(
        _gather_body,
        out_type=jax.ShapeDtypeStruct((_H, _W), jnp.float32),
        mesh=plsc.VectorSubcoreMesh(core_axis_name="c", subcore_axis_name="s"),
        compiler_params=pltpu.CompilerParams(needs_layout_passes=False),
        scratch_types=[
            pltpu.VMEM((_PADH,), jnp.int32),
            pltpu.VMEM((_BPW,), jnp.int32),
            pltpu.VMEM((_BPW,), jnp.float32),
            pltpu.SemaphoreType.DMA,
        ],
    )
    return gk(table, pixel_to_node)


# speedup vs baseline: 1.6104x; 1.0064x over previous
import jax
import jax.numpy as jnp
from jax import lax
from jax.experimental import pallas as pl
from jax.experimental.pallas import tpu as pltpu
from jax.experimental.pallas import tpu_sc as plsc

_NUM_NODES = 200000
_H = 512
_W = 512
_BETA_F = 100.0

_PADH = 100352
_ROWS = _PADH // 128
_GRID = 2
_BLK = _ROWS // _GRID
_NC, _NS = 2, 16
_NW = _NC * _NS
_B = _H * _W
_BPW = _B // _NW
_RPW = _H // _NW
_LANES = 16


def _pack_table_body(t1, t2, t3, l1, l2, l3, llv, h1, h2, h3, hlv, out):
    def nv(a1, a2, a3, lv):
        m = jnp.minimum(
            jnp.minimum(a1[...] - t1[0, 0], a2[...] - t2[0, 0]),
            a3[...] - t3[0, 0],
        )
        v = jax.nn.sigmoid(_BETA_F * m) * lv[...]
        bits = lax.bitcast_convert_type(v, jnp.int32)
        return (bits + 0x7FFF + ((bits >> 16) & 1)) >> 16

    out[...] = nv(l1, l2, l3, llv) | (nv(h1, h2, h3, hlv) << 16)


def _gather_body(table, idx, out, table_v, idx_v, vals_v, sem):
    wid = lax.axis_index("s") * _NC + lax.axis_index("c")

    @pl.when(wid < _NW // 2)
    def _active():
        chunk = _PADH // 8
        copies = [
            pltpu.make_async_copy(table.at[pl.ds(k * chunk, chunk)],
                                  table_v.at[pl.ds(k * chunk, chunk)], sem)
            for k in range(8)
        ]
        for c in copies:
            c.start()

        def idx_copy(sl):
            return pltpu.make_async_copy(idx.at[pl.ds(sl * _BPW, _BPW)],
                                         idx_v, sem)

        def out_copies(sl):
            return [
                pltpu.make_async_copy(vals_v.at[pl.ds(r * _W, _W)],
                                      out.at[sl * _RPW + r, :], sem)
                for r in range(_RPW)
            ]

        def gather_loop():
            @plsc.parallel_loop(0, _BPW // _LANES, 1, unroll=16)
            def _gather(i):
                off = i * _LANES
                iv = idx_v[pl.ds(off, _LANES)]
                hi = iv >= _PADH
                word_idx = iv - jnp.where(hi, _PADH, 0)
                w = plsc.load_gather(table_v, [word_idx])
                fbits = (w >> jnp.where(hi, 16, 0)) << 16
                vals_v[pl.ds(off, _LANES)] = plsc.bitcast(fbits, jnp.float32)

        sl_a = wid
        sl_b = wid + _NW // 2
        ca = idx_copy(sl_a)
        ca.start()
        for c in copies:
            c.wait()
        ca.wait()
        gather_loop()
        outs_a = out_copies(sl_a)
        for c in outs_a:
            c.start()
        cb = idx_copy(sl_b)
        cb.start()
        for c in outs_a:
            c.wait()
        cb.wait()
        gather_loop()
        outs_b = out_copies(sl_b)
        for c in outs_b:
            c.start()
        for c in outs_b:
            c.wait()


def kernel(a_scaled_1, a_scaled_2, a_scaled_3, thr_1, thr_2, thr_3,
           node_levels, pixel_to_node):
    def prep(x):
        return jnp.pad(x, (0, 2 * _PADH - _NUM_NODES)).reshape(2 * _ROWS, 128)

    a1 = prep(a_scaled_1)
    a2 = prep(a_scaled_2)
    a3 = prep(a_scaled_3)
    lv = prep(node_levels)
    t1 = thr_1.reshape(1, 1)
    t2 = thr_2.reshape(1, 1)
    t3 = thr_3.reshape(1, 1)

    smem = pl.BlockSpec((1, 1), lambda i: (0, 0), memory_space=pltpu.SMEM)
    lo = pl.BlockSpec((_BLK, 128), lambda i: (i, 0))
    hi = pl.BlockSpec((_BLK, 128), lambda i: (_GRID + i, 0))
    table = pl.pallas_call(
        _pack_table_body,
        grid=(_GRID,),
        out_shape=jax.ShapeDtypeStruct((_ROWS, 128), jnp.int32),
        in_specs=[smem, smem, smem, lo, lo, lo, lo, hi, hi, hi, hi],
        out_specs=pl.BlockSpec((_BLK, 128), lambda i: (i, 0)),
    )(t1, t2, t3, a1, a2, a3, lv, a1, a2, a3, lv).reshape(-1)

    gk = pl.kernel(
        _gather_body,
        out_type=jax.ShapeDtypeStruct((_H, _W), jnp.float32),
        mesh=plsc.VectorSubcoreMesh(core_axis_name="c", subcore_axis_name="s"),
        compiler_params=pltpu.CompilerParams(needs_layout_passes=False),
        scratch_types=[
            pltpu.VMEM((_PADH,), jnp.int32),
            pltpu.VMEM((_BPW,), jnp.int32),
            pltpu.VMEM((_BPW,), jnp.float32),
            pltpu.SemaphoreType.DMA,
        ],
    )
    return gk(table, pixel_to_node)


# speedup vs baseline: 1.6627x; 1.0324x over previous
import jax
import jax.numpy as jnp
from jax import lax
from jax.experimental import pallas as pl
from jax.experimental.pallas import tpu as pltpu
from jax.experimental.pallas import tpu_sc as plsc

_NUM_NODES = 200000
_H = 512
_W = 512
_BETA_F = 100.0

_PADH = 100352
_ROWS = _PADH // 128
_GRID = 2
_BLK = _ROWS // _GRID
_NC, _NS = 2, 16
_NW = _NC * _NS
_B = _H * _W
_BPW = _B // _NW
_RPW = _H // _NW
_LANES = 16


def _pack_table_body(t1, t2, t3, a1, a2, a3, lv, out):
    m = jnp.minimum(
        jnp.minimum(a1[...] - t1[0, 0], a2[...] - t2[0, 0]),
        a3[...] - t3[0, 0],
    )
    v = jax.nn.sigmoid(_BETA_F * m) * lv[...]
    bits = lax.bitcast_convert_type(v, jnp.int32)
    r = (bits + 0x7FFF + ((bits >> 16) & 1)) >> 16

    step = pl.program_id(0)

    @pl.when(step == 0)
    def _lo():
        out[...] = r

    @pl.when(step == 1)
    def _hi():
        out[...] = out[...] | (r << 16)


def _gather_body(table, idx, out, table_v, idx_v, vals_v, sem):
    wid = lax.axis_index("s") * _NC + lax.axis_index("c")

    @pl.when(wid < _NW // 2)
    def _active():
        chunk = _PADH // 8
        copies = [
            pltpu.make_async_copy(table.at[pl.ds(k * chunk, chunk)],
                                  table_v.at[pl.ds(k * chunk, chunk)], sem)
            for k in range(8)
        ]
        for c in copies:
            c.start()

        def idx_copy(sl):
            return pltpu.make_async_copy(idx.at[pl.ds(sl * _BPW, _BPW)],
                                         idx_v, sem)

        def out_copies(sl):
            return [
                pltpu.make_async_copy(vals_v.at[pl.ds(r * _W, _W)],
                                      out.at[sl * _RPW + r, :], sem)
                for r in range(_RPW)
            ]

        def gather_loop():
            @plsc.parallel_loop(0, _BPW // _LANES, 1, unroll=16)
            def _gather(i):
                off = i * _LANES
                iv = idx_v[pl.ds(off, _LANES)]
                hi = iv >= _PADH
                word_idx = iv - jnp.where(hi, _PADH, 0)
                w = plsc.load_gather(table_v, [word_idx])
                fbits = (w >> jnp.where(hi, 16, 0)) << 16
                vals_v[pl.ds(off, _LANES)] = plsc.bitcast(fbits, jnp.float32)

        sl_a = wid
        sl_b = wid + _NW // 2
        ca = idx_copy(sl_a)
        ca.start()
        for c in copies:
            c.wait()
        ca.wait()
        gather_loop()
        outs_a = out_copies(sl_a)
        for c in outs_a:
            c.start()
        cb = idx_copy(sl_b)
        cb.start()
        for c in outs_a:
            c.wait()
        cb.wait()
        gather_loop()
        outs_b = out_copies(sl_b)
        for c in outs_b:
            c.start()
        for c in outs_b:
            c.wait()


def kernel(a_scaled_1, a_scaled_2, a_scaled_3, thr_1, thr_2, thr_3,
           node_levels, pixel_to_node):
    t1 = thr_1.reshape(1, 1)
    t2 = thr_2.reshape(1, 1)
    t3 = thr_3.reshape(1, 1)

    smem = pl.BlockSpec((1, 1), lambda i: (0, 0), memory_space=pltpu.SMEM)
    blk = pl.BlockSpec((_PADH,), lambda i: (i,))
    table = pl.pallas_call(
        _pack_table_body,
        grid=(2,),
        out_shape=jax.ShapeDtypeStruct((_PADH,), jnp.int32),
        in_specs=[smem, smem, smem, blk, blk, blk, blk],
        out_specs=pl.BlockSpec((_PADH,), lambda i: (0,)),
    )(t1, t2, t3, a_scaled_1, a_scaled_2, a_scaled_3, node_levels)

    gk = pl.kernel(
        _gather_body,
        out_type=jax.ShapeDtypeStruct((_H, _W), jnp.float32),
        mesh=plsc.VectorSubcoreMesh(core_axis_name="c", subcore_axis_name="s"),
        compiler_params=pltpu.CompilerParams(needs_layout_passes=False),
        scratch_types=[
            pltpu.VMEM((_PADH,), jnp.int32),
            pltpu.VMEM((_BPW,), jnp.int32),
            pltpu.VMEM((_BPW,), jnp.float32),
            pltpu.SemaphoreType.DMA,
        ],
    )
    return gk(table, pixel_to_node)
